# Initial kernel scaffold; baseline (speedup 1.0000x reference)
import jax
import jax.numpy as jnp
from jax.experimental import pallas as pl


def kernel(edge_emb_eq1, edge_index, Wl1, bl1, Wr1, Wl2, bl2, Wr2, Ws, bs, Wm, bm, Wv, bv):
    raise NotImplementedError("write your pallas kernel here")


# speedup vs baseline: 7.8971x; 7.8971x over previous
import functools

import jax
import jax.numpy as jnp
from jax import lax
from jax.experimental import pallas as pl
from jax.experimental.pallas import tpu as pltpu
from jax.experimental.pallas import tpu_sc as plsc

_N = 10000
_E = 160000
_D = 256
_H = 128
_NT = 16
_NC = 2
_CH = 80
_EPT = _E // _NT
_NIT = _EPT // _CH
_RPT = 632
_RLAST = _N - 15 * _RPT
_RB = 1000
_G = _N // _RB


def _agg_body(x_hbm, srcx_hbm, dst_hbm, z_hbm, out_hbm,
              sidx, didx, s80, d80, rows, acc, sem):
    c = lax.axis_index("c")
    s = lax.axis_index("s")
    pltpu.sync_copy(srcx_hbm.at[c * _NT + s], sidx)
    pltpu.sync_copy(dst_hbm.at[s], didx)
    @pl.when(s < _NT - 1)
    def _():
        pltpu.sync_copy(z_hbm, acc.at[pl.ds(s * _RPT, _RPT)])

    @pl.when(s == _NT - 1)
    def _():
        pltpu.sync_copy(z_hbm.at[pl.ds(0, _RLAST)],
                        acc.at[pl.ds(15 * _RPT, _RLAST)])

    plsc.subcore_barrier()

    def step(j, carry):
        for k in range(_CH // 16):
            s80[pl.ds(k * 16, 16)] = sidx[j, pl.ds(k * 16, 16)]
            d80[pl.ds(k * 16, 16)] = didx[j, pl.ds(k * 16, 16)]
        pltpu.async_copy(x_hbm.at[s80], rows, sem).wait()
        pltpu.sync_copy(rows, acc.at[d80], add=True)
        return carry

    lax.fori_loop(0, _NIT, step, 0)
    plsc.subcore_barrier()

    @pl.when(s < _NT - 1)
    def _():
        pltpu.sync_copy(acc.at[pl.ds(s * _RPT, _RPT)],
                        out_hbm.at[pl.ds(c * _N + s * _RPT, _RPT)])

    @pl.when(s == _NT - 1)
    def _():
        pltpu.sync_copy(acc.at[pl.ds(15 * _RPT, _RLAST)],
                        out_hbm.at[pl.ds(c * _N + 15 * _RPT, _RLAST)])


@functools.cache
def _make_agg(interpret: bool = False):
    return pl.kernel(
        _agg_body,
        out_type=jax.ShapeDtypeStruct((_NC * _N, _H), jnp.float32),
        mesh=plsc.VectorSubcoreMesh(core_axis_name="c", subcore_axis_name="s"),
        scratch_types=[
            pltpu.VMEM((_NIT, _CH), jnp.int32),
            pltpu.VMEM((_NIT, _CH), jnp.int32),
            pltpu.VMEM((_CH,), jnp.int32),
            pltpu.VMEM((_CH,), jnp.int32),
            pltpu.VMEM((_CH, _H), jnp.float32),
            pltpu.VMEM_SHARED((_N, _H), jnp.float32),
            pltpu.SemaphoreType.DMA,
        ],
        interpret=interpret,
    )


def _s1_body(s_ref, x_ref, c_ref, wl_ref, wr_ref, b_ref, o_ref):
    ic = 1.0 / jnp.maximum(c_ref[...], 1.0)
    S = jnp.concatenate([s_ref[0], s_ref[1]], axis=1)
    X = jnp.concatenate([x_ref[0], x_ref[1]], axis=1)
    h = (jnp.dot(S * ic, wl_ref[...], preferred_element_type=jnp.float32)
         + jnp.dot(X, wr_ref[...], preferred_element_type=jnp.float32)
         + b_ref[...])
    h = jnp.maximum(h, 0.0)
    o_ref[0] = h[:, :_H]
    o_ref[1] = h[:, _H:]


def _s2_body(s_ref, x_ref, c_ref, wl_ref, wr_ref, b_ref, u_ref, h_ref):
    cnt = c_ref[...]
    ic = 1.0 / jnp.maximum(cnt, 1.0)
    dinv = lax.rsqrt(cnt + 1.0)
    S = jnp.concatenate([s_ref[0], s_ref[1]], axis=1)
    X = jnp.concatenate([x_ref[0], x_ref[1]], axis=1)
    h = (jnp.dot(S * ic, wl_ref[...], preferred_element_type=jnp.float32)
         + jnp.dot(X, wr_ref[...], preferred_element_type=jnp.float32)
         + b_ref[...])
    u = dinv * h
    u_ref[0] = u[:, :_H]
    u_ref[1] = u[:, _H:]
    h_ref[0] = h[:, :_H]
    h_ref[1] = h[:, _H:]


def _s3_body(g_ref, x_ref, c_ref, ws_ref, b_ref, v_ref, o_ref):
    dinv = lax.rsqrt(c_ref[...] + 1.0)
    Gm = jnp.concatenate([g_ref[0], g_ref[1]], axis=1)
    X = jnp.concatenate([x_ref[0], x_ref[1]], axis=1)
    P = dinv * Gm + (dinv * dinv) * X
    xs = jnp.maximum(
        jnp.dot(P, ws_ref[...], preferred_element_type=jnp.float32) + b_ref[...],
        0.0)
    v = dinv * xs
    v_ref[0] = v[:, :_H]
    v_ref[1] = v[:, _H:]
    o_ref[0] = xs[:, :_H]
    o_ref[1] = xs[:, _H:]


def _s4_body(g_ref, x_ref, c_ref, wm_ref, bm_ref, wv_ref, bv_ref, mu_ref, lv_ref):
    dinv = lax.rsqrt(c_ref[...] + 1.0)
    Gm = jnp.concatenate([g_ref[0], g_ref[1]], axis=1)
    X = jnp.concatenate([x_ref[0], x_ref[1]], axis=1)
    t = dinv * Gm + (dinv * dinv) * X
    mu_ref[...] = jnp.dot(t, wm_ref[...], preferred_element_type=jnp.float32) + bm_ref[...]
    lv_ref[...] = jnp.dot(t, wv_ref[...], preferred_element_type=jnp.float32) + bv_ref[...]


_FEAT = pl.BlockSpec((_NC, _RB, _H), lambda i: (0, i, 0))
_COL = pl.BlockSpec((_RB, 1), lambda i: (i, 0))
_W = pl.BlockSpec((_D, _D), lambda i: (0, 0))
_B = pl.BlockSpec((1, _D), lambda i: (0, 0))
_FULL = pl.BlockSpec((_RB, _D), lambda i: (i, 0))
_FSHAPE = jax.ShapeDtypeStruct((_NC, _N, _H), jnp.float32)
_OSHAPE = jax.ShapeDtypeStruct((_N, _D), jnp.float32)


@functools.cache
def _make_stages(interpret: bool = False):
    s1 = pl.pallas_call(
        _s1_body, grid=(_G,),
        in_specs=[_FEAT, _FEAT, _COL, _W, _W, _B],
        out_specs=_FEAT, out_shape=_FSHAPE, interpret=interpret)
    s2 = pl.pallas_call(
        _s2_body, grid=(_G,),
        in_specs=[_FEAT, _FEAT, _COL, _W, _W, _B],
        out_specs=(_FEAT, _FEAT), out_shape=(_FSHAPE, _FSHAPE),
        interpret=interpret)
    s3 = pl.pallas_call(
        _s3_body, grid=(_G,),
        in_specs=[_FEAT, _FEAT, _COL, _W, _B],
        out_specs=(_FEAT, _FEAT), out_shape=(_FSHAPE, _FSHAPE),
        interpret=interpret)
    s4 = pl.pallas_call(
        _s4_body, grid=(_G,),
        in_specs=[_FEAT, _FEAT, _COL, _W, _B, _W, _B],
        out_specs=(_FULL, _FULL), out_shape=(_OSHAPE, _OSHAPE),
        interpret=interpret)
    return s1, s2, s3, s4


def _run(x, edge_index, Wl1, bl1, Wr1, Wl2, bl2, Wr2, Ws, bs, Wm, bm, Wv, bv,
         interpret: bool = False):
    agg = _make_agg(interpret)
    s1, s2, s3, s4 = _make_stages(interpret)

    src = edge_index[0]
    dst = edge_index[1]
    srcx = jnp.concatenate([src, src + _N]).reshape(_NC * _NT, _NIT, _CH)
    dst2 = dst.reshape(_NT, _NIT, _CH)
    zblk = jnp.zeros((_RPT, _H), jnp.float32)
    ones = jnp.ones((_NC * _N, _H), jnp.float32)

    X2 = jnp.moveaxis(x.reshape(_N, 2, _H), 1, 0)

    cnt = agg(ones, srcx, dst2, zblk)[:_N, :1]
    S1 = agg(X2.reshape(_NC * _N, _H), srcx, dst2, zblk).reshape(_NC, _N, _H)
    H1 = s1(S1, X2, cnt, Wl1.T, Wr1.T, bl1.reshape(1, _D))
    S2 = agg(H1.reshape(_NC * _N, _H), srcx, dst2, zblk).reshape(_NC, _N, _H)
    U, H2 = s2(S2, H1, cnt, Wl2.T, Wr2.T, bl2.reshape(1, _D))
    Gm = agg(U.reshape(_NC * _N, _H), srcx, dst2, zblk).reshape(_NC, _N, _H)
    V, XS = s3(Gm, H2, cnt, Ws.T, bs.reshape(1, _D))
    Hh = agg(V.reshape(_NC * _N, _H), srcx, dst2, zblk).reshape(_NC, _N, _H)
    mu, lv = s4(Hh, XS, cnt, Wm.T, bm.reshape(1, _D), Wv.T, bv.reshape(1, _D))
    return (mu, lv)


def kernel(edge_emb_eq1, edge_index, Wl1, bl1, Wr1, Wl2, bl2, Wr2, Ws, bs,
           Wm, bm, Wv, bv):
    return _run(edge_emb_eq1, edge_index, Wl1, bl1, Wr1, Wl2, bl2, Wr2,
                Ws, bs, Wm, bm, Wv, bv)
